# baseline (speedup 1.0000x reference)
import functools

import jax
import jax.numpy as jnp
from jax import lax
from jax.experimental import pallas as pl
from jax.experimental.pallas import tpu as pltpu
from jax.experimental.pallas import tpu_sc as plsc

_B, _H, _W, _D = 8, 512, 512, 19
_BSC = 4
_NC, _NS, _L = 2, 16, 16
_NW = _NC * _NS
_WPB = _NW // _BSC
_RPW = _H // _WPB
_SR, _SC = 8, 256
_NSUB = (_RPW // _SR) * (_W // _SC)
_CPS = _W // _SC


def _sc_body(x_hbm, out_hbm, x_v0, x_v1, o_v0, o_v1,
             sem_i0, sem_i1, sem_o0, sem_o1):
    wid = lax.axis_index("s") * _NC + lax.axis_index("c")
    b = wid // _WPB
    row0 = (wid % _WPB) * _RPW
    bufs = ((x_v0, o_v0, sem_i0, sem_o0), (x_v1, o_v1, sem_i1, sem_o1))

    def rc(s):
        return row0 + (s // _CPS) * _SR, (s % _CPS) * _SC

    def in_slice(s):
        r, c = rc(s)
        return x_hbm.at[b, pl.ds(r, _SR), pl.ds(c, _SC)]

    def out_slice(s):
        r, c = rc(s)
        return out_hbm.at[b, :, pl.ds(r, _SR), pl.ds(c, _SC)]

    def compute(x_v, o_v):
        def vec(i, carry):
            for srow in range(_SR):
                x = x_v[srow, pl.ds(i * _L, _L)]
                for d in range(_D):
                    o_v[d, srow, pl.ds(i * _L, _L)] = jnp.where(
                        x == d, jnp.float32(1.0), jnp.float32(0.0))
            return carry

        lax.fori_loop(0, _SC // _L, vec, 0)

    pltpu.sync_copy(in_slice(0), x_v0)
    pltpu.async_copy(in_slice(1), x_v1, sem_i1)
    compute(x_v0, o_v0)
    pltpu.async_copy(o_v0, out_slice(0), sem_o0)
    pltpu.async_copy(in_slice(2), x_v0, sem_i0)
    pltpu.make_async_copy(in_slice(1), x_v1, sem_i1).wait()
    compute(x_v1, o_v1)
    pltpu.async_copy(o_v1, out_slice(1), sem_o1)
    pltpu.async_copy(in_slice(3), x_v1, sem_i1)

    def pair(p, carry):
        for k, (x_v, o_v, sem_i, sem_o) in enumerate(bufs):
            s = 2 * p + k
            pltpu.make_async_copy(in_slice(s), x_v, sem_i).wait()
            pltpu.make_async_copy(o_v, out_slice(s - 2), sem_o).wait()
            compute(x_v, o_v)
            pltpu.async_copy(o_v, out_slice(s), sem_o)

            @pl.when(s < _NSUB - 2)
            def _():
                pltpu.async_copy(in_slice(s + 2), x_v, sem_i)

        return carry

    lax.fori_loop(1, _NSUB // 2, pair, 0)

    for k, (x_v, o_v, sem_i, sem_o) in enumerate(bufs):
        pltpu.make_async_copy(o_v, out_slice(_NSUB - 2 + k), sem_o).wait()


@jax.jit
def _one_hot_sc(x):
    mesh = plsc.VectorSubcoreMesh(core_axis_name="c", subcore_axis_name="s")
    f = functools.partial(
        pl.kernel,
        out_type=jax.ShapeDtypeStruct((_BSC, _D, _H, _W), jnp.float32),
        mesh=mesh,
        compiler_params=pltpu.CompilerParams(use_tc_tiling_on_sc=True),
        scratch_types=[
            pltpu.VMEM((_SR, _SC), jnp.int32),
            pltpu.VMEM((_SR, _SC), jnp.int32),
            pltpu.VMEM((_D, _SR, _SC), jnp.float32),
            pltpu.VMEM((_D, _SR, _SC), jnp.float32),
            pltpu.SemaphoreType.DMA,
            pltpu.SemaphoreType.DMA,
            pltpu.SemaphoreType.DMA,
            pltpu.SemaphoreType.DMA,
        ],
    )(_sc_body)
    return f(x)


_RB = 64


def _tc_body(x_ref, o_ref):
    x = x_ref[0]
    d = lax.broadcasted_iota(jnp.int32, (_D, _RB, _W), 0)
    o_ref[0] = jnp.where(x[None] == d, jnp.float32(1.0), jnp.float32(0.0))


@jax.jit
def _one_hot_tc(x):
    nb = x.shape[0]
    return pl.pallas_call(
        _tc_body,
        grid=(nb, _H // _RB),
        in_specs=[pl.BlockSpec((1, _RB, _W), lambda b, r: (b, r, 0))],
        out_specs=pl.BlockSpec((1, _D, _RB, _W), lambda b, r: (b, 0, r, 0)),
        out_shape=jax.ShapeDtypeStruct((nb, _D, _H, _W), jnp.float32),
    )(x)


def kernel(X_in, ones):
    del ones
    lo = _one_hot_sc(X_in[:_BSC])
    hi = _one_hot_tc(X_in[_BSC:])
    return jnp.concatenate([lo, hi], axis=0)


# speedup vs baseline: 2.2384x; 2.2384x over previous
import functools

import jax
import jax.numpy as jnp
from jax import lax
from jax.experimental import pallas as pl
from jax.experimental.pallas import tpu as pltpu
from jax.experimental.pallas import tpu_sc as plsc

_B, _H, _W, _D = 8, 512, 512, 19
_NC, _NS, _L = 2, 16, 16
_NW = _NC * _NS
_WPB = _NW // _B
_RPW = _H // _WPB
_SR, _SC = 8, 256
_NSUB = (_RPW // _SR) * (_W // _SC)
_CPS = _W // _SC


def _sc_body(x_hbm, out_hbm, x_v0, x_v1, o_v0, o_v1,
             sem_i0, sem_i1, sem_o0, sem_o1):
    wid = lax.axis_index("s") * _NC + lax.axis_index("c")
    b = wid // _WPB
    row0 = (wid % _WPB) * _RPW
    bufs = ((x_v0, o_v0, sem_i0, sem_o0), (x_v1, o_v1, sem_i1, sem_o1))

    def rc(s):
        return row0 + (s // _CPS) * _SR, (s % _CPS) * _SC

    def in_slice(s):
        r, c = rc(s)
        return x_hbm.at[b, pl.ds(r, _SR), pl.ds(c, _SC)]

    def out_slice(s):
        r, c = rc(s)
        return out_hbm.at[b, :, pl.ds(r, _SR), pl.ds(c, _SC)]

    def compute(x_v, o_v):
        def vec(i, carry):
            for srow in range(_SR):
                x = x_v[srow, pl.ds(i * _L, _L)]
                for d in range(_D):
                    o_v[d, srow, pl.ds(i * _L, _L)] = jnp.where(
                        x == d, jnp.float32(1.0), jnp.float32(0.0))
            return carry

        lax.fori_loop(0, _SC // _L, vec, 0)

    pltpu.sync_copy(in_slice(0), x_v0)
    pltpu.async_copy(in_slice(1), x_v1, sem_i1)
    compute(x_v0, o_v0)
    pltpu.async_copy(o_v0, out_slice(0), sem_o0)
    pltpu.async_copy(in_slice(2), x_v0, sem_i0)
    pltpu.make_async_copy(in_slice(1), x_v1, sem_i1).wait()
    compute(x_v1, o_v1)
    pltpu.async_copy(o_v1, out_slice(1), sem_o1)
    pltpu.async_copy(in_slice(3), x_v1, sem_i1)

    def pair(p, carry):
        for k, (x_v, o_v, sem_i, sem_o) in enumerate(bufs):
            s = 2 * p + k
            pltpu.make_async_copy(in_slice(s), x_v, sem_i).wait()
            pltpu.make_async_copy(o_v, out_slice(s - 2), sem_o).wait()
            compute(x_v, o_v)
            pltpu.async_copy(o_v, out_slice(s), sem_o)

            @pl.when(s < _NSUB - 2)
            def _():
                pltpu.async_copy(in_slice(s + 2), x_v, sem_i)

        return carry

    lax.fori_loop(1, _NSUB // 2, pair, 0)

    for k, (x_v, o_v, sem_i, sem_o) in enumerate(bufs):
        pltpu.make_async_copy(o_v, out_slice(_NSUB - 2 + k), sem_o).wait()


@jax.jit
def _one_hot_sc(x):
    mesh = plsc.VectorSubcoreMesh(core_axis_name="c", subcore_axis_name="s")
    f = functools.partial(
        pl.kernel,
        out_type=jax.ShapeDtypeStruct((_B, _D, _H, _W), jnp.float32),
        mesh=mesh,
        compiler_params=pltpu.CompilerParams(use_tc_tiling_on_sc=True),
        scratch_types=[
            pltpu.VMEM((_SR, _SC), jnp.int32),
            pltpu.VMEM((_SR, _SC), jnp.int32),
            pltpu.VMEM((_D, _SR, _SC), jnp.float32),
            pltpu.VMEM((_D, _SR, _SC), jnp.float32),
            pltpu.SemaphoreType.DMA,
            pltpu.SemaphoreType.DMA,
            pltpu.SemaphoreType.DMA,
            pltpu.SemaphoreType.DMA,
        ],
    )(_sc_body)
    return f(x)


def kernel(X_in, ones):
    del ones
    return _one_hot_sc(X_in)
